# Initial kernel scaffold; baseline (speedup 1.0000x reference)
import jax
import jax.numpy as jnp
from jax.experimental import pallas as pl


def kernel(x, edge_index, edge_attr, batch_ids, embed_table, hq_w, fck_w1, fck_w2, fcv_w1, fcv_w2, dot_w, lin_w, lin_b, wp_w, wp_b):
    raise NotImplementedError("write your pallas kernel here")


# speedup vs baseline: 1.7046x; 1.7046x over previous
import functools
import math

import jax
import jax.numpy as jnp
from jax import lax
from jax.experimental import pallas as pl
from jax.experimental.pallas import tpu as pltpu
from jax.experimental.pallas import tpu_sc as plsc

N_NODES = 10000
N_EDGES = 80000
H = 32
NB = 10
MAX_R = 2.0
N_GRAPHS = 500
C_SILU = 1.6790

E_PAD = 81920
N_PAD = 10240
G_PAD = 512
X_PAD = 10240
EB = 1280
STEP = MAX_R / (NB + 1)


def _sc_info():
    info = plsc.get_sparse_core_info()
    return info.num_cores, info.num_subcores


def _gather_rows(table, idx, b_pad, chunk):
    nc, ns = _sc_info()
    nw = nc * ns
    b_per_w = b_pad // nw
    n_chunks = b_per_w // chunk
    d = table.shape[1]

    mesh = plsc.VectorSubcoreMesh(core_axis_name="c", subcore_axis_name="s")

    @functools.partial(
        pl.kernel,
        mesh=mesh,
        out_type=jax.ShapeDtypeStruct((b_pad, d), jnp.float32),
        scratch_types=[
            pltpu.VMEM((chunk,), jnp.int32),
            pltpu.VMEM((chunk, d), jnp.float32),
            pltpu.SemaphoreType.DMA,
        ],
        compiler_params=pltpu.CompilerParams(use_tc_tiling_on_sc=False),
    )
    def k(table_hbm, idx_hbm, out_hbm, idx_v, rows_v, sem):
        wid = lax.axis_index("s") * nc + lax.axis_index("c")
        base = wid * b_per_w

        def body(j, carry):
            off = base + j * chunk
            pltpu.sync_copy(idx_hbm.at[pl.ds(off, chunk)], idx_v)
            pltpu.async_copy(table_hbm.at[idx_v], rows_v, sem).wait()
            pltpu.sync_copy(rows_v, out_hbm.at[pl.ds(off, chunk)])
            return carry

        lax.fori_loop(0, n_chunks, body, 0)

    return k(table, idx)


def _scatter_add_rows(vals, idx, zeros, n_pad, chunk):
    nc, ns = _sc_info()
    nw = nc * ns
    e_pad = vals.shape[0]
    d = vals.shape[1]
    e_per_w = e_pad // nw
    n_chunks = e_per_w // chunk
    rows_per_sub = n_pad // ns

    mesh = plsc.VectorSubcoreMesh(core_axis_name="c", subcore_axis_name="s")

    @functools.partial(
        pl.kernel,
        mesh=mesh,
        out_type=jax.ShapeDtypeStruct((nc * n_pad, d), jnp.float32),
        scratch_types=[
            pltpu.VMEM_SHARED((n_pad, d), jnp.float32),
            pltpu.VMEM((chunk, d), jnp.float32),
            pltpu.VMEM((chunk,), jnp.int32),
        ],
        compiler_params=pltpu.CompilerParams(use_tc_tiling_on_sc=False),
    )
    def k(vals_hbm, idx_hbm, zeros_hbm, out_hbm, shared, vals_v, idx_v):
        cid = lax.axis_index("c")
        sid = lax.axis_index("s")
        wid = sid * nc + cid
        zoff = sid * rows_per_sub
        pltpu.sync_copy(zeros_hbm.at[pl.ds(zoff, rows_per_sub)],
                        shared.at[pl.ds(zoff, rows_per_sub)])
        plsc.subcore_barrier()
        base = wid * e_per_w

        def body(j, carry):
            off = base + j * chunk
            pltpu.sync_copy(vals_hbm.at[pl.ds(off, chunk)], vals_v)
            pltpu.sync_copy(idx_hbm.at[pl.ds(off, chunk)], idx_v)
            pltpu.sync_copy(vals_v, shared.at[idx_v], add=True)
            return carry

        lax.fori_loop(0, n_chunks, body, 0)
        plsc.subcore_barrier()
        pltpu.sync_copy(shared.at[pl.ds(zoff, rows_per_sub)],
                        out_hbm.at[pl.ds(cid * n_pad + zoff, rows_per_sub)])

    return k(vals, idx, zeros)


def _sus(y):
    safe = jnp.where(y > 0.0, y, 1.0)
    return jnp.where(y > 0.0, jnp.exp(-1.0 / safe), 0.0)


def _edge_body(ea_ref, fs_ref, qd_ref, w1_ref, w2_ref, out_ref):
    ea = ea_ref[...]
    elen = jnp.sqrt(jnp.sum(ea * ea, axis=1, keepdims=True) + 1e-24)
    i16i = lax.broadcasted_iota(jnp.int32, (EB, 16), 1)
    i16 = i16i.astype(jnp.float32)
    diff = elen / STEP - (i16 + 1.0)
    s_raw = 1.14136 * math.exp(2.0) * _sus(diff + 1.0) * _sus(1.0 - diff)
    s_raw = jnp.where(i16i < NB, s_raw, 0.0)
    cutoff = _sus(10.0 * (1.0 - elen / MAX_R))

    a = jnp.dot(s_raw, w1_ref[...], preferred_element_type=jnp.float32)
    h = a * (1.0 / (1.0 + jnp.exp(-a))) * C_SILU

    fs = fs_ref[...]
    t = jnp.dot(fs, w2_ref[...], preferred_element_type=jnp.float32)

    kk = jnp.zeros((EB, H), jnp.float32)
    vv = jnp.zeros((EB, H), jnp.float32)
    for j in range(16):
        kk = kk + h[:, j:j + 1] * t[:, j * H:(j + 1) * H]
        vv = vv + h[:, 16 + j:17 + j] * t[:, 512 + j * H:512 + (j + 1) * H]

    d = jnp.sum(qd_ref[...] * kk, axis=1, keepdims=True)
    expd = cutoff * jnp.exp(d)
    out_ref[...] = jnp.sqrt(expd) * vv


def _edge_pass(ea_pad, f_src, qd_dst, w1cat, w2kv):
    grid = E_PAD // EB
    return pl.pallas_call(
        _edge_body,
        grid=(grid,),
        in_specs=[
            pl.BlockSpec((EB, 8), lambda i: (i, 0)),
            pl.BlockSpec((EB, H), lambda i: (i, 0)),
            pl.BlockSpec((EB, H), lambda i: (i, 0)),
            pl.BlockSpec((16, H), lambda i: (0, 0)),
            pl.BlockSpec((H, 1024), lambda i: (0, 0)),
        ],
        out_specs=pl.BlockSpec((EB, H), lambda i: (i, 0)),
        out_shape=jax.ShapeDtypeStruct((E_PAD, H), jnp.float32),
    )(ea_pad, f_src, qd_dst, w1cat, w2kv)


def _init_body(f0_ref, hqd_ref, out_ref):
    f0 = f0_ref[0:N_NODES, :]
    out_ref[0:N_NODES, :] = f0
    out_ref[N_NODES:2 * N_NODES, :] = jnp.dot(
        f0, hqd_ref[...], preferred_element_type=jnp.float32)


def _init_pass(f0_pad, hqd0):
    return pl.pallas_call(
        _init_body,
        out_shape=jax.ShapeDtypeStruct((2 * N_NODES, H), jnp.float32),
    )(f0_pad, hqd0)


def _node_norm(u_ref):
    u = u_ref[0:N_NODES, :] + u_ref[N_PAD:N_PAD + N_NODES, :]
    nrm = jnp.sqrt(jnp.sum(u * u, axis=1, keepdims=True) + 1e-24)
    h_x = u / jnp.maximum(nrm, 1e-12)
    return jnp.maximum(h_x, 0.0)


def _mid_body(u_ref, f_ref, hqd_ref, out_ref):
    f1 = _node_norm(u_ref) + f_ref[0:N_NODES, :]
    out_ref[0:N_NODES, :] = f1
    out_ref[N_NODES:2 * N_NODES, :] = jnp.dot(
        f1, hqd_ref[...], preferred_element_type=jnp.float32)


def _mid_pass(u, fq, hqd1):
    return pl.pallas_call(
        _mid_body,
        out_shape=jax.ShapeDtypeStruct((2 * N_NODES, H), jnp.float32),
    )(u, fq, hqd1)


def _out_body(u_ref, f_ref, l0_ref, b0_ref, l1_ref, b1_ref, out_ref):
    f2 = _node_norm(u_ref) + f_ref[0:N_NODES, :]
    y = jnp.maximum(
        jnp.dot(f2, l0_ref[...], preferred_element_type=jnp.float32)
        + b0_ref[...], 0.0)
    y = jnp.maximum(
        jnp.dot(y, l1_ref[...], preferred_element_type=jnp.float32)
        + b1_ref[...], 0.0)
    out_ref[0:N_NODES, :] = y
    out_ref[N_NODES:X_PAD, :] = jnp.zeros((X_PAD - N_NODES, H), jnp.float32)


def _out_pass(u, fq, l0, b0, l1, b1):
    return pl.pallas_call(
        _out_body,
        out_shape=jax.ShapeDtypeStruct((X_PAD, H), jnp.float32),
    )(u, fq, l0, b0, l1, b1)


def _proj_body(g_ref, wp_ref, bp_ref, out_ref):
    m = g_ref[0:N_GRAPHS, :] + g_ref[G_PAD:G_PAD + N_GRAPHS, :]
    out_ref[...] = jnp.dot(
        m, wp_ref[...], preferred_element_type=jnp.float32) + bp_ref[...]


def _proj_pass(g, wp_w, wp_b):
    return pl.pallas_call(
        _proj_body,
        out_shape=jax.ShapeDtypeStruct((N_GRAPHS, wp_w.shape[1]), jnp.float32),
    )(g, wp_w, wp_b)


def kernel(x, edge_index, edge_attr, batch_ids, embed_table, hq_w, fck_w1,
           fck_w2, fcv_w1, fcv_w2, dot_w, lin_w, lin_b, wp_w, wp_b):
    src = edge_index[0].astype(jnp.int32)
    dst = edge_index[1].astype(jnp.int32)

    pad_i = E_PAD - N_EDGES
    idx2 = jnp.concatenate([
        src, jnp.zeros((pad_i,), jnp.int32),
        dst + N_NODES, jnp.zeros((pad_i,), jnp.int32)])
    dst_pad = jnp.concatenate(
        [dst, jnp.full((pad_i,), N_PAD - 1, jnp.int32)])
    x_pad = jnp.concatenate(
        [x.astype(jnp.int32), jnp.zeros((X_PAD - N_NODES,), jnp.int32)])
    bat_pad = jnp.concatenate([
        batch_ids.astype(jnp.int32),
        jnp.full((X_PAD - N_NODES,), G_PAD - 1, jnp.int32)])

    ea_pad = jnp.zeros((E_PAD, 8), jnp.float32)
    ea_pad = ea_pad.at[:N_EDGES, :3].set(edge_attr.astype(jnp.float32))

    zeros_n = jnp.zeros((N_PAD, H), jnp.float32)
    zeros_g = jnp.zeros((G_PAD, H), jnp.float32)

    sc_kv = 1.0 / (4.0 * math.sqrt(float(H)))
    hqd = [hq_w[m] @ dot_w[m] / (float(H) * math.sqrt(float(H)))
           for m in range(2)]
    w1cat = [jnp.concatenate(
        [jnp.pad(fck_w1[m], ((0, 6), (0, 0))),
         jnp.pad(fcv_w1[m], ((0, 6), (0, 0)))], axis=1) for m in range(2)]
    w2kv = [jnp.concatenate(
        [fck_w2[m].reshape(16, H, H).transpose(1, 0, 2).reshape(H, 512),
         fcv_w2[m].reshape(16, H, H).transpose(1, 0, 2).reshape(H, 512)],
        axis=1) * sc_kv for m in range(2)]

    f0_pad = _gather_rows(embed_table.astype(jnp.float32), x_pad, X_PAD, 64)
    fq = _init_pass(f0_pad, hqd[0])

    for m in range(2):
        g = _gather_rows(fq, idx2, 2 * E_PAD, 128)
        f_src = g[:E_PAD]
        qd_dst = g[E_PAD:]
        w_e = _edge_pass(ea_pad, f_src, qd_dst, w1cat[m], w2kv[m])
        u = _scatter_add_rows(w_e, dst_pad, zeros_n, N_PAD, 128)
        if m == 0:
            fq = _mid_pass(u, fq, hqd[1])
        else:
            y = _out_pass(u, fq, lin_w[0], lin_b[0][None, :],
                          lin_w[1], lin_b[1][None, :])

    gsum = _scatter_add_rows(y, bat_pad, zeros_g, G_PAD, 64)
    return _proj_pass(gsum, wp_w, wp_b[None, :])
